# baseline (speedup 1.0000x reference)
import functools

import jax
import jax.numpy as jnp
from jax import lax
from jax.experimental import pallas as pl
from jax.experimental.pallas import tpu as pltpu
from jax.experimental.pallas import tpu_sc as plsc

N_NODES = 10000
N_EDGES = 320000
IN_CH = 128
HIDDEN = 32
ACT = 20
N_EDGE_PAT = 40
BB = 20
RPB = BB * ACT
OPB = BB * N_EDGE_PAT
GRID = (N_NODES // ACT) // BB

NW = 32
EPW = N_EDGES // NW
CH = 80
NP = 10240
RPT = NP // 16
NCHUNK = EPW // CH
NRING = 4
NOUTER = (NCHUNK - 1) // NRING

@functools.cache
def _sc_mesh():
    return plsc.VectorSubcoreMesh(core_axis_name="c", subcore_axis_name="s",
                                  num_cores=2, num_subcores=16)


def _deg_body(dst_hbm, out_hbm, dstbuf, degbuf):
    wid = lax.axis_index("s") * 2 + lax.axis_index("c")

    def zero(i, _):
        degbuf[pl.ds(i * 16, 16)] = jnp.zeros((16,), jnp.float32)
        return 0

    lax.fori_loop(0, N_NODES // 16, zero, 0)
    pltpu.sync_copy(dst_hbm.at[pl.ds(wid * EPW, EPW)], dstbuf)
    ones = jnp.ones((16,), jnp.float32)

    def body(i, _):
        idx = dstbuf[pl.ds(i * 16, 16)]
        plsc.addupdate_scatter(degbuf, [idx], ones)
        return 0

    lax.fori_loop(0, EPW // 16, body, 0)
    for j in range(10):
        pltpu.sync_copy(degbuf.at[pl.ds(j * 1000, 1000)],
                        out_hbm.at[pl.ds(j * (NW * 1000) + wid * 1000, 1000)])


@functools.cache
def _deg_partials_kernel():
    return pl.kernel(
        _deg_body,
        out_type=jax.ShapeDtypeStruct((10 * NW * (N_NODES // 10),), jnp.float32),
        scratch_types=[
            pltpu.VMEM((EPW,), jnp.int32),
            pltpu.VMEM((N_NODES,), jnp.float32),
        ],
        mesh=_sc_mesh(),
        compiler_params=pltpu.CompilerParams(needs_layout_passes=False),
    )


def _mm_body(x_ref, w_ref, degp_ref, xwn_ref, dinv_ref):
    deg = jnp.sum(degp_ref[0], axis=0) + 1.0
    dinv = lax.rsqrt(deg)[:, None]
    xw = jnp.dot(x_ref[...], w_ref[...], preferred_element_type=jnp.float32)
    xwn_ref[...] = xw * dinv
    dinv_ref[...] = dinv


@jax.jit
def _matmul_scale(state, w, degp):
    return pl.pallas_call(
        _mm_body,
        grid=(10,),
        in_specs=[
            pl.BlockSpec((N_NODES // 10, IN_CH), lambda i: (i, 0)),
            pl.BlockSpec((IN_CH, IN_CH), lambda i: (0, 0)),
            pl.BlockSpec((1, NW, N_NODES // 10), lambda i: (i, 0, 0)),
        ],
        out_specs=[
            pl.BlockSpec((N_NODES // 10, IN_CH), lambda i: (i, 0)),
            pl.BlockSpec((N_NODES // 10, 1), lambda i: (i, 0)),
        ],
        out_shape=[
            jax.ShapeDtypeStruct((N_NODES, IN_CH), jnp.float32),
            jax.ShapeDtypeStruct((N_NODES, 1), jnp.float32),
        ],
    )(state, w, degp)


def _scat_body(xwn_hbm, src_hbm, dst_hbm, zero_hbm, out0_hbm, out1_hbm,
               srcc0, srcc1, srcc2, srcc3, dstc0, dstc1, dstc2, dstc3,
               rows0, rows1, rows2, rows3, accum, gsems):
    cid = lax.axis_index("c")
    sid = lax.axis_index("s")
    wid = sid * 2 + cid
    rows = [rows0, rows1, rows2, rows3]
    srcc = [srcc0, srcc1, srcc2, srcc3]
    dstc = [dstc0, dstc1, dstc2, dstc3]
    pltpu.sync_copy(zero_hbm.at[pl.ds(sid * RPT, RPT)],
                    accum.at[pl.ds(sid * RPT, RPT)])
    plsc.subcore_barrier()

    def fetch(j, b):
        base = wid * EPW + j * CH
        pltpu.sync_copy(src_hbm.at[pl.ds(base, CH)], srcc[b])
        pltpu.async_copy(xwn_hbm.at[srcc[b]], rows[b], gsems.at[b])
        pltpu.sync_copy(dst_hbm.at[pl.ds(base, CH)], dstc[b])

    for b in range(NRING):
        fetch(b, b)

    def outer(g, _):
        for b in range(NRING):
            j = g * NRING + b
            pltpu.make_async_copy(xwn_hbm.at[srcc[b]], rows[b],
                                  gsems.at[b]).wait()
            pltpu.sync_copy(rows[b], accum.at[dstc[b]], add=True)

            @pl.when(j + NRING < NCHUNK)
            def _():
                fetch(j + NRING, b)

        return 0

    lax.fori_loop(0, NOUTER, outer, 0)
    pltpu.make_async_copy(xwn_hbm.at[srcc[0]], rows[0], gsems.at[0]).wait()
    pltpu.sync_copy(rows[0], accum.at[dstc[0]], add=True)
    plsc.subcore_barrier()

    @pl.when(cid == 0)
    def _():
        pltpu.sync_copy(accum.at[pl.ds(sid * RPT, RPT)],
                        out0_hbm.at[pl.ds(sid * RPT, RPT)])

    @pl.when(cid == 1)
    def _():
        pltpu.sync_copy(accum.at[pl.ds(sid * RPT, RPT)],
                        out1_hbm.at[pl.ds(sid * RPT, RPT)])


@functools.cache
def _scatter_rows_kernel():
    return pl.kernel(
        _scat_body,
        out_type=[jax.ShapeDtypeStruct((NP, IN_CH), jnp.float32),
                  jax.ShapeDtypeStruct((NP, IN_CH), jnp.float32)],
        scratch_types=(
            [pltpu.VMEM((CH,), jnp.int32)] * (2 * NRING)
            + [pltpu.VMEM((CH, IN_CH), jnp.float32)] * NRING
            + [pltpu.VMEM_SHARED((NP, IN_CH), jnp.float32),
               pltpu.SemaphoreType.DMA((NRING,))]
        ),
        mesh=_sc_mesh(),
    )


def _head_body(acc0_ref, acc1_ref, xwn_ref, dinv_ref, state_ref, bgcn_ref,
               w1t_ref, w1b_ref, b1_ref, w2_ref, b2_ref, wmu_ref, bmu_ref,
               ks_ref, kd_ref, out_ref):
    accsum = acc0_ref[...] + acc1_ref[...] + xwn_ref[...]
    x2 = accsum * dinv_ref[...] + bgcn_ref[...]
    x2 = jnp.maximum(x2, 0.0) + state_ref[...]
    p = jnp.dot(x2, w1t_ref[...], preferred_element_type=jnp.float32)
    q = jnp.dot(x2, w1b_ref[...], preferred_element_type=jnp.float32)
    h1 = (jnp.dot(ks_ref[...], p, preferred_element_type=jnp.float32)
          + jnp.dot(kd_ref[...], q, preferred_element_type=jnp.float32)
          + b1_ref[...])
    h1 = jnp.where(h1 >= 0.0, h1, 0.01 * h1)
    h2 = jnp.dot(h1, w2_ref[...], preferred_element_type=jnp.float32) + b2_ref[...]
    h2 = jnp.where(h2 >= 0.0, h2, 0.01 * h2)
    z = jnp.dot(h2, wmu_ref[...], preferred_element_type=jnp.float32) + bmu_ref[...] + 1e-10
    out_ref[...] = jnp.maximum(z, 0.0) + jnp.log1p(jnp.exp(-jnp.abs(z)))


@jax.jit
def _head(acc0, acc1, xwn, dinv, state, b_gcn, w1t, w1b, b1, w2, b2,
          wmu, bmu, ks, kd):
    full = lambda s: pl.BlockSpec(s, lambda i: (0,) * len(s))
    row = pl.BlockSpec((RPB, IN_CH), lambda i: (i, 0))
    return pl.pallas_call(
        _head_body,
        grid=(GRID,),
        in_specs=[
            row, row, row,
            pl.BlockSpec((RPB, 1), lambda i: (i, 0)),
            row,
            full((1, IN_CH)),
            full((IN_CH, HIDDEN)),
            full((IN_CH, HIDDEN)),
            full((1, HIDDEN)),
            full((HIDDEN, HIDDEN)),
            full((1, HIDDEN)),
            full((HIDDEN, 1)),
            full((1, 1)),
            full((OPB, RPB)),
            full((OPB, RPB)),
        ],
        out_specs=pl.BlockSpec((OPB, 1), lambda i: (i, 0)),
        out_shape=jax.ShapeDtypeStruct((GRID * OPB, 1), jnp.float32),
    )(acc0, acc1, xwn, dinv, state, b_gcn, w1t, w1b, b1, w2, b2,
      wmu, bmu, ks, kd)


def kernel(state, edge_index, edges, W_gcn, b_gcn, W_l1, b_l1, W_l2, b_l2,
           W_mu, b_mu, W_sig, b_sig):
    src = edge_index[0]
    dst = edge_index[1]
    degp = _deg_partials_kernel()(dst).reshape(10, NW, N_NODES // 10)
    xwn, dinv = _matmul_scale(state, W_gcn, degp)
    zeros = jnp.zeros((NP, IN_CH), jnp.float32)
    acc0, acc1 = _scatter_rows_kernel()(xwn, src, dst, zeros)
    sel_s = jax.nn.one_hot(edges[:, 0], ACT, dtype=jnp.float32)
    sel_d = jax.nn.one_hot(edges[:, 1], ACT, dtype=jnp.float32)
    eye = jnp.eye(BB, dtype=jnp.float32)
    ks = jnp.kron(eye, sel_s)
    kd = jnp.kron(eye, sel_d)
    mu = _head(acc0, acc1, xwn, dinv, state, b_gcn[None, :],
               W_l1[:IN_CH], W_l1[IN_CH:], b_l1[None, :],
               W_l2, b_l2[None, :], W_mu, b_mu[None, :], ks, kd)
    return mu.reshape(N_NODES // ACT, N_EDGE_PAT)


# speedup vs baseline: 1.0087x; 1.0087x over previous
import functools

import jax
import jax.numpy as jnp
from jax import lax
from jax.experimental import pallas as pl
from jax.experimental.pallas import tpu as pltpu
from jax.experimental.pallas import tpu_sc as plsc

N_NODES = 10000
N_EDGES = 320000
IN_CH = 128
HIDDEN = 32
ACT = 20
N_EDGE_PAT = 40
BB = 20
RPB = BB * ACT
OPB = BB * N_EDGE_PAT
GRID = (N_NODES // ACT) // BB

NW = 32
EPW = N_EDGES // NW
CH = 80
NP = 10240
RPT = NP // 16
NCHUNK = EPW // CH
NRING = 4
NOUTER = (NCHUNK - 1) // NRING

@functools.cache
def _sc_mesh():
    return plsc.VectorSubcoreMesh(core_axis_name="c", subcore_axis_name="s",
                                  num_cores=2, num_subcores=16)


def _deg_body(dst_hbm, out_hbm, dstbuf, degbuf):
    wid = lax.axis_index("s") * 2 + lax.axis_index("c")

    def zero(i, _):
        degbuf[pl.ds(i * 16, 16)] = jnp.zeros((16,), jnp.float32)
        return 0

    lax.fori_loop(0, N_NODES // 16, zero, 0)
    pltpu.sync_copy(dst_hbm.at[pl.ds(wid * EPW, EPW)], dstbuf)
    ones = jnp.ones((16,), jnp.float32)

    def body(i, _):
        idx = dstbuf[pl.ds(i * 16, 16)]
        plsc.addupdate_scatter(degbuf, [idx], ones)
        return 0

    lax.fori_loop(0, EPW // 16, body, 0)
    for j in range(10):
        pltpu.sync_copy(degbuf.at[pl.ds(j * 1000, 1000)],
                        out_hbm.at[pl.ds(j * (NW * 1000) + wid * 1000, 1000)])


@functools.cache
def _deg_partials_kernel():
    return pl.kernel(
        _deg_body,
        out_type=jax.ShapeDtypeStruct((10 * NW * (N_NODES // 10),), jnp.float32),
        scratch_types=[
            pltpu.VMEM((EPW,), jnp.int32),
            pltpu.VMEM((N_NODES,), jnp.float32),
        ],
        mesh=_sc_mesh(),
        compiler_params=pltpu.CompilerParams(needs_layout_passes=False),
    )


def _mm_body(x_ref, w_ref, degp_ref, xwn_ref, dinv_ref):
    deg = jnp.sum(degp_ref[0], axis=0) + 1.0
    dinv = lax.rsqrt(deg)[:, None]
    xw = jnp.dot(x_ref[...], w_ref[...], preferred_element_type=jnp.float32)
    xwn_ref[...] = xw * dinv
    dinv_ref[...] = dinv


@jax.jit
def _matmul_scale(state, w, degp):
    return pl.pallas_call(
        _mm_body,
        grid=(10,),
        in_specs=[
            pl.BlockSpec((N_NODES // 10, IN_CH), lambda i: (i, 0)),
            pl.BlockSpec((IN_CH, IN_CH), lambda i: (0, 0)),
            pl.BlockSpec((1, NW, N_NODES // 10), lambda i: (i, 0, 0)),
        ],
        out_specs=[
            pl.BlockSpec((N_NODES // 10, IN_CH), lambda i: (i, 0)),
            pl.BlockSpec((N_NODES // 10, 1), lambda i: (i, 0)),
        ],
        out_shape=[
            jax.ShapeDtypeStruct((N_NODES, IN_CH), jnp.float32),
            jax.ShapeDtypeStruct((N_NODES, 1), jnp.float32),
        ],
    )(state, w, degp)


def _scat_body(xwn_hbm, src_hbm, dst_hbm, zero_hbm, out0_hbm, out1_hbm,
               srcc0, srcc1, srcc2, srcc3, dstc0, dstc1, dstc2, dstc3,
               rows0, rows1, rows2, rows3, accum, gsems):
    cid = lax.axis_index("c")
    sid = lax.axis_index("s")
    wid = sid * 2 + cid
    rows = [rows0, rows1, rows2, rows3]
    srcc = [srcc0, srcc1, srcc2, srcc3]
    dstc = [dstc0, dstc1, dstc2, dstc3]
    @pl.when((cid == 0) & (sid < 15))
    def _():
        pltpu.sync_copy(xwn_hbm.at[pl.ds(sid * RPT, RPT)],
                        accum.at[pl.ds(sid * RPT, RPT)])

    @pl.when((cid == 0) & (sid == 15))
    def _():
        pltpu.sync_copy(xwn_hbm.at[pl.ds(15 * RPT, N_NODES - 15 * RPT)],
                        accum.at[pl.ds(15 * RPT, N_NODES - 15 * RPT)])
        pltpu.sync_copy(zero_hbm.at[pl.ds(0, NP - N_NODES)],
                        accum.at[pl.ds(N_NODES, NP - N_NODES)])

    @pl.when(cid == 1)
    def _():
        pltpu.sync_copy(zero_hbm, accum.at[pl.ds(sid * RPT, RPT)])

    plsc.subcore_barrier()

    def fetch(j, b):
        base = wid * EPW + j * CH
        pltpu.sync_copy(src_hbm.at[pl.ds(base, CH)], srcc[b])
        pltpu.async_copy(xwn_hbm.at[srcc[b]], rows[b], gsems.at[b])
        pltpu.sync_copy(dst_hbm.at[pl.ds(base, CH)], dstc[b])

    for b in range(NRING):
        fetch(b, b)

    def outer(g, _):
        for b in range(NRING):
            j = g * NRING + b
            pltpu.make_async_copy(xwn_hbm.at[srcc[b]], rows[b],
                                  gsems.at[b]).wait()
            pltpu.sync_copy(rows[b], accum.at[dstc[b]], add=True)

            @pl.when(j + NRING < NCHUNK)
            def _():
                fetch(j + NRING, b)

        return 0

    lax.fori_loop(0, NOUTER, outer, 0)
    pltpu.make_async_copy(xwn_hbm.at[srcc[0]], rows[0], gsems.at[0]).wait()
    pltpu.sync_copy(rows[0], accum.at[dstc[0]], add=True)
    plsc.subcore_barrier()

    @pl.when(cid == 0)
    def _():
        pltpu.sync_copy(accum.at[pl.ds(sid * RPT, RPT)],
                        out0_hbm.at[pl.ds(sid * RPT, RPT)])

    @pl.when(cid == 1)
    def _():
        pltpu.sync_copy(accum.at[pl.ds(sid * RPT, RPT)],
                        out1_hbm.at[pl.ds(sid * RPT, RPT)])


@functools.cache
def _scatter_rows_kernel():
    return pl.kernel(
        _scat_body,
        out_type=[jax.ShapeDtypeStruct((NP, IN_CH), jnp.float32),
                  jax.ShapeDtypeStruct((NP, IN_CH), jnp.float32)],
        scratch_types=(
            [pltpu.VMEM((CH,), jnp.int32)] * (2 * NRING)
            + [pltpu.VMEM((CH, IN_CH), jnp.float32)] * NRING
            + [pltpu.VMEM_SHARED((NP, IN_CH), jnp.float32),
               pltpu.SemaphoreType.DMA((NRING,))]
        ),
        mesh=_sc_mesh(),
    )


def _head_body(acc0_ref, acc1_ref, dinv_ref, state_ref, bgcn_ref,
               w1t_ref, w1b_ref, b1_ref, w2_ref, b2_ref, wmu_ref, bmu_ref,
               ks_ref, kd_ref, out_ref):
    accsum = acc0_ref[...] + acc1_ref[...]
    x2 = accsum * dinv_ref[...] + bgcn_ref[...]
    x2 = jnp.maximum(x2, 0.0) + state_ref[...]
    p = jnp.dot(x2, w1t_ref[...], preferred_element_type=jnp.float32)
    q = jnp.dot(x2, w1b_ref[...], preferred_element_type=jnp.float32)
    h1 = (jnp.dot(ks_ref[...], p, preferred_element_type=jnp.float32)
          + jnp.dot(kd_ref[...], q, preferred_element_type=jnp.float32)
          + b1_ref[...])
    h1 = jnp.where(h1 >= 0.0, h1, 0.01 * h1)
    h2 = jnp.dot(h1, w2_ref[...], preferred_element_type=jnp.float32) + b2_ref[...]
    h2 = jnp.where(h2 >= 0.0, h2, 0.01 * h2)
    z = jnp.dot(h2, wmu_ref[...], preferred_element_type=jnp.float32) + bmu_ref[...] + 1e-10
    out_ref[...] = jnp.maximum(z, 0.0) + jnp.log1p(jnp.exp(-jnp.abs(z)))


@jax.jit
def _head(acc0, acc1, dinv, state, b_gcn, w1t, w1b, b1, w2, b2,
          wmu, bmu, ks, kd):
    full = lambda s: pl.BlockSpec(s, lambda i: (0,) * len(s))
    row = pl.BlockSpec((RPB, IN_CH), lambda i: (i, 0))
    return pl.pallas_call(
        _head_body,
        grid=(GRID,),
        in_specs=[
            row, row,
            pl.BlockSpec((RPB, 1), lambda i: (i, 0)),
            row,
            full((1, IN_CH)),
            full((IN_CH, HIDDEN)),
            full((IN_CH, HIDDEN)),
            full((1, HIDDEN)),
            full((HIDDEN, HIDDEN)),
            full((1, HIDDEN)),
            full((HIDDEN, 1)),
            full((1, 1)),
            full((OPB, RPB)),
            full((OPB, RPB)),
        ],
        out_specs=pl.BlockSpec((OPB, 1), lambda i: (i, 0)),
        out_shape=jax.ShapeDtypeStruct((GRID * OPB, 1), jnp.float32),
    )(acc0, acc1, dinv, state, b_gcn, w1t, w1b, b1, w2, b2,
      wmu, bmu, ks, kd)


def kernel(state, edge_index, edges, W_gcn, b_gcn, W_l1, b_l1, W_l2, b_l2,
           W_mu, b_mu, W_sig, b_sig):
    src = edge_index[0]
    dst = edge_index[1]
    degp = _deg_partials_kernel()(dst).reshape(10, NW, N_NODES // 10)
    xwn, dinv = _matmul_scale(state, W_gcn, degp)
    zeros = jnp.zeros((RPT, IN_CH), jnp.float32)
    acc0, acc1 = _scatter_rows_kernel()(xwn, src, dst, zeros)
    sel_s = jax.nn.one_hot(edges[:, 0], ACT, dtype=jnp.float32)
    sel_d = jax.nn.one_hot(edges[:, 1], ACT, dtype=jnp.float32)
    eye = jnp.eye(BB, dtype=jnp.float32)
    ks = jnp.kron(eye, sel_s)
    kd = jnp.kron(eye, sel_d)
    mu = _head(acc0, acc1, dinv, state, b_gcn[None, :],
               W_l1[:IN_CH], W_l1[IN_CH:], b_l1[None, :],
               W_l2, b_l2[None, :], W_mu, b_mu[None, :], ks, kd)
    return mu.reshape(N_NODES // ACT, N_EDGE_PAT)
